# baseline (speedup 1.0000x reference)
import functools

import jax
import jax.numpy as jnp
from jax import lax
from jax.experimental import pallas as pl
from jax.experimental.pallas import tpu as pltpu
from jax.experimental.pallas import tpu_sc as plsc

_WIDTH = 128
_NFEAT = 16
_N = 10000
_E = 320000

_NW = 32
_EPW = _E // _NW


_GSUB = 80
_GROWS = 8
_GCHUNK = _GSUB * _GROWS
_NCHUNK = _E // _GCHUNK


def _sc_diffs(x, xneg, e0r, e1r):
    mesh = plsc.VectorSubcoreMesh(core_axis_name="c", subcore_axis_name="s")

    @functools.partial(
        pl.kernel,
        mesh=mesh,
        compiler_params=pltpu.CompilerParams(needs_layout_passes=False),
        out_type=jax.ShapeDtypeStruct((_E, _WIDTH), jnp.float32),
        scratch_types=[
            pltpu.VMEM((_GROWS, _GSUB), jnp.int32),
            pltpu.VMEM((_GROWS, _GSUB), jnp.int32),
            pltpu.VMEM((_GCHUNK, _WIDTH), jnp.float32),
            pltpu.SemaphoreType.DMA,
        ],
    )
    def k(x_hbm, xn_hbm, e0_hbm, e1_hbm, out_hbm, i0, i1, buf, sem):
        wid = lax.axis_index("s") * 2 + lax.axis_index("c")
        iters = (_NCHUNK + _NW - 1) // _NW

        def chunk(t, _):
            c = wid + t * _NW

            @pl.when(c < _NCHUNK)
            def _():
                row0 = c * _GROWS
                pltpu.sync_copy(e0_hbm.at[pl.ds(row0, _GROWS)], i0)
                pltpu.sync_copy(e1_hbm.at[pl.ds(row0, _GROWS)], i1)
                for kk in range(_GROWS):
                    pltpu.async_copy(
                        x_hbm.at[i1.at[kk]], buf.at[pl.ds(kk * _GSUB, _GSUB)], sem
                    )
                for kk in range(_GROWS):
                    pltpu.make_async_copy(
                        x_hbm.at[i1.at[kk]], buf.at[pl.ds(kk * _GSUB, _GSUB)], sem
                    ).wait()
                for kk in range(_GROWS):
                    pltpu.async_copy(
                        xn_hbm.at[i0.at[kk]],
                        buf.at[pl.ds(kk * _GSUB, _GSUB)],
                        sem,
                        add=True,
                    )
                for kk in range(_GROWS):
                    pltpu.make_async_copy(
                        xn_hbm.at[i0.at[kk]], buf.at[pl.ds(kk * _GSUB, _GSUB)], sem
                    ).wait()
                pltpu.sync_copy(buf, out_hbm.at[pl.ds(c * _GCHUNK, _GCHUNK)])

            return 0

        lax.fori_loop(0, iters, chunk, 0)

    return k(x, xneg, e0r, e1r)


_ERB = 2560


def _tc_edge(diffs, e_feat, w1, w2, b):
    def body(d_ref, f_ref, w1_ref, w2_ref, b_ref, o_ref):
        d = d_ref[...]
        h = jnp.dot(d, w1_ref[...], preferred_element_type=jnp.float32)
        h = h + jnp.dot(f_ref[...], w2_ref[...], preferred_element_type=jnp.float32)
        h = h + b_ref[...]
        o_ref[...] = d + jnp.maximum(h, 0.0)

    return pl.pallas_call(
        body,
        grid=(_E // _ERB,),
        in_specs=[
            pl.BlockSpec((_ERB, _WIDTH), lambda i: (i, 0)),
            pl.BlockSpec((_ERB, _NFEAT), lambda i: (i, 0)),
            pl.BlockSpec((_WIDTH, _WIDTH), lambda i: (0, 0)),
            pl.BlockSpec((_NFEAT, _WIDTH), lambda i: (0, 0)),
            pl.BlockSpec((1, _WIDTH), lambda i: (0, 0)),
        ],
        out_specs=pl.BlockSpec((_ERB, _WIDTH), lambda i: (i, 0)),
        out_shape=jax.ShapeDtypeStruct((_E, _WIDTH), jnp.float32),
    )(diffs, e_feat, w1, w2, b)


_OWN = 320
_NPAD = _OWN * _NW
_SCHUNK = 3200
_SVEC = _SCHUNK // 16
_SROWS = _SCHUNK // 16 // 5
_BATCH = 256
_NEG = float("-inf")


def _sc_segmax(e1, ef):
    mesh = plsc.VectorSubcoreMesh(core_axis_name="c", subcore_axis_name="s")

    @functools.partial(
        pl.kernel,
        mesh=mesh,
        compiler_params=pltpu.CompilerParams(needs_layout_passes=False),
        out_type=jax.ShapeDtypeStruct((_NPAD, _WIDTH), jnp.float32),
        scratch_types=[
            pltpu.VMEM((_SCHUNK,), jnp.int32),
            pltpu.VMEM((_SCHUNK + 96,), jnp.int32),
            pltpu.VMEM((2, 128), jnp.int32),
            pltpu.VMEM((_BATCH,), jnp.int32),
            pltpu.VMEM((_BATCH, _WIDTH), jnp.float32),
            pltpu.VMEM((_OWN, _WIDTH), jnp.float32),
            pltpu.VMEM((16,), jnp.int32),
            pltpu.VMEM_SHARED((_SCHUNK, _WIDTH), jnp.float32),
            pltpu.SemaphoreType.DMA,
        ],
    )
    def k(e1_hbm, ef_hbm, out_hbm, idxc, mylist, idsb, lib, rowbuf, acc,
          tmp16, spbuf, sem):
        sid = lax.axis_index("s")
        wid = sid * 2 + lax.axis_index("c")
        lo = wid * _OWN
        iota16 = lax.iota(jnp.int32, 16)
        neg = jnp.full((16,), _NEG, jnp.float32)

        def init_acc(i, _):
            for c in range(8):
                acc[i, pl.ds(c * 16, 16)] = neg
            return 0

        lax.fori_loop(0, _OWN, init_acc, 0)

        zero16 = jnp.zeros((16,), jnp.int32)

        def init_list(j, _):
            mylist[pl.ds(j * 16, 16)] = zero16
            return 0

        lax.fori_loop(0, (_SCHUNK + 96) // 16, init_list, 0)

        def chunk(t, _):
            plsc.subcore_barrier()
            for h in range(5):
                pltpu.async_copy(
                    ef_hbm.at[
                        pl.ds(t * _SCHUNK + (sid * 5 + h) * _SROWS, _SROWS)
                    ],
                    spbuf.at[pl.ds((sid * 5 + h) * _SROWS, _SROWS)],
                    sem,
                )
            pltpu.sync_copy(e1_hbm.at[pl.ds(t * _SCHUNK, _SCHUNK)], idxc)

            def scanb(j, cv):
                v = idxc[pl.ds(j * 16, 16)]
                m = (v >= lo) & (v < lo + _OWN)
                inc = jnp.cumsum(jnp.where(m, zero16 + 1, zero16))
                pos = cv + inc - 1
                eid = j * 16 + iota16
                packed = jnp.bitwise_or(lax.shift_left(eid, 9), v - lo)
                plsc.store_scatter(mylist, [pos], packed, mask=m)
                return cv + plsc.all_reduce_population_count(m)

            cnt_vec = lax.fori_loop(0, _SVEC, scanb, zero16)
            cnt = jnp.max(cnt_vec)
            for h in range(5):
                pltpu.make_async_copy(
                    ef_hbm.at[
                        pl.ds(t * _SCHUNK + (sid * 5 + h) * _SROWS, _SROWS)
                    ],
                    spbuf.at[pl.ds((sid * 5 + h) * _SROWS, _SROWS)],
                    sem,
                ).wait()
            plsc.subcore_barrier()

            def sub(s, _):
                for g in range(16):
                    pk = mylist[pl.ds(s * _BATCH + g * 16, 16)]
                    idsb[g // 8, pl.ds((g % 8) * 16, 16)] = (
                        lax.shift_right_logical(pk, 9)
                    )
                    lib[pl.ds(g * 16, 16)] = jnp.bitwise_and(pk, 511)
                pltpu.async_copy(
                    spbuf.at[idsb.at[0]], rowbuf.at[pl.ds(0, 128)], sem
                )
                pltpu.async_copy(
                    spbuf.at[idsb.at[1]], rowbuf.at[pl.ds(128, 128)], sem
                )
                pltpu.make_async_copy(
                    spbuf.at[idsb.at[0]], rowbuf.at[pl.ds(0, 128)], sem
                ).wait()
                pltpu.make_async_copy(
                    spbuf.at[idsb.at[1]], rowbuf.at[pl.ds(128, 128)], sem
                ).wait()
                nrows = jnp.clip(cnt - s * _BATCH, 0, _BATCH)
                ngroups = lax.shift_right_logical(nrows + 15, 4)

                def grp(g, _):
                    base = g * 16
                    li = lib[pl.ds(base, 16)]
                    li_s, perm = plsc.sort_key_val(li, iota16)
                    tmp16[pl.ds(0, 16)] = li_s
                    prev = plsc.load_gather(
                        tmp16, [jnp.maximum(iota16 - 1, 0)]
                    )
                    is_start = (li_s != prev) | (iota16 == 0)
                    run_start = plsc.cummax(
                        jnp.where(is_start, iota16, zero16)
                    )
                    run_pos = iota16 - run_start
                    valid = perm < (nrows - base)
                    nr = jnp.max(jnp.where(valid, run_pos, zero16)) + 1
                    rowi = base + perm

                    def rnd(r, _):
                        mr = (run_pos == r) & valid
                        for j in range(_WIDTH):
                            colj = zero16 + j
                            av = plsc.load_gather(acc, [li_s, colj], mask=mr)
                            rv = plsc.load_gather(rowbuf, [rowi, colj], mask=mr)
                            plsc.store_scatter(
                                acc, [li_s, colj], jnp.maximum(av, rv), mask=mr
                            )
                        return 0

                    lax.fori_loop(0, nr, rnd, 0)
                    return 0

                lax.fori_loop(0, ngroups, grp, 0)

            def sub_guard(s, _):
                pl.when(s * _BATCH < cnt)(lambda: sub(s, 0))
                return 0

            lax.fori_loop(0, _SCHUNK // _BATCH + 1, sub_guard, 0)
            return 0

        lax.fori_loop(0, _E // _SCHUNK, chunk, 0)

        zf = jnp.zeros((16,), jnp.float32)

        def fini(i, _):
            for c in range(8):
                v = acc[i, pl.ds(c * 16, 16)]
                acc[i, pl.ds(c * 16, 16)] = jnp.where(v == _NEG, zf, v)
            return 0

        lax.fori_loop(0, _OWN, fini, 0)
        pltpu.sync_copy(acc, out_hbm.at[pl.ds(lo, _OWN)])

    return k(e1, ef)


_NRB = 2000


def _tc_node(x, maxes, m1, m2, b):
    def body(x_ref, mx_ref, m1_ref, m2_ref, b_ref, o_ref):
        xv = x_ref[...]
        h = jnp.dot(xv, m1_ref[...], preferred_element_type=jnp.float32)
        h = h + jnp.dot(mx_ref[...], m2_ref[...], preferred_element_type=jnp.float32)
        h = h + b_ref[...]
        o_ref[...] = xv + jnp.maximum(h, 0.0)

    return pl.pallas_call(
        body,
        grid=(_N // _NRB,),
        in_specs=[
            pl.BlockSpec((_NRB, _WIDTH), lambda i: (i, 0)),
            pl.BlockSpec((_NRB, _WIDTH), lambda i: (i, 0)),
            pl.BlockSpec((_WIDTH, _WIDTH), lambda i: (0, 0)),
            pl.BlockSpec((_WIDTH, _WIDTH), lambda i: (0, 0)),
            pl.BlockSpec((1, _WIDTH), lambda i: (0, 0)),
        ],
        out_specs=pl.BlockSpec((_NRB, _WIDTH), lambda i: (i, 0)),
        out_shape=jax.ShapeDtypeStruct((_N, _WIDTH), jnp.float32),
    )(x, maxes, m1, m2, b)


def kernel(x, e, e_feat, edge_w, edge_b, mlp_w, mlp_b):
    e32 = e.astype(jnp.int32)
    e0r = e32[0].reshape(_E // _GSUB, _GSUB)
    e1r = e32[1].reshape(_E // _GSUB, _GSUB)
    diffs = _sc_diffs(x, -x, e0r, e1r)
    ef = _tc_edge(
        diffs, e_feat, edge_w[:_WIDTH], edge_w[_WIDTH:], edge_b.reshape(1, _WIDTH)
    )
    maxes = _sc_segmax(e32[1], ef)[:_N]
    return _tc_node(
        x, maxes, mlp_w[:_WIDTH], mlp_w[_WIDTH:], mlp_b.reshape(1, _WIDTH)
    )


# speedup vs baseline: 1.0428x; 1.0428x over previous
import functools

import jax
import jax.numpy as jnp
from jax import lax
from jax.experimental import pallas as pl
from jax.experimental.pallas import tpu as pltpu
from jax.experimental.pallas import tpu_sc as plsc

_WIDTH = 128
_NFEAT = 16
_N = 10000
_E = 320000

_NW = 32
_EPW = _E // _NW


_GSUB = 80
_GROWS = 8
_GCHUNK = _GSUB * _GROWS
_NCHUNK = _E // _GCHUNK


def _sc_diffs(x, xneg, e0r, e1r):
    mesh = plsc.VectorSubcoreMesh(core_axis_name="c", subcore_axis_name="s")

    @functools.partial(
        pl.kernel,
        mesh=mesh,
        compiler_params=pltpu.CompilerParams(needs_layout_passes=False),
        out_type=jax.ShapeDtypeStruct((_E, _WIDTH), jnp.float32),
        scratch_types=[
            pltpu.VMEM((_GROWS, _GSUB), jnp.int32),
            pltpu.VMEM((_GROWS, _GSUB), jnp.int32),
            pltpu.VMEM((_GCHUNK, _WIDTH), jnp.float32),
            pltpu.SemaphoreType.DMA,
        ],
    )
    def k(x_hbm, xn_hbm, e0_hbm, e1_hbm, out_hbm, i0, i1, buf, sem):
        wid = lax.axis_index("s") * 2 + lax.axis_index("c")
        iters = (_NCHUNK + _NW - 1) // _NW

        def chunk(t, _):
            c = wid + t * _NW

            @pl.when(c < _NCHUNK)
            def _():
                row0 = c * _GROWS
                pltpu.sync_copy(e0_hbm.at[pl.ds(row0, _GROWS)], i0)
                pltpu.sync_copy(e1_hbm.at[pl.ds(row0, _GROWS)], i1)
                for kk in range(_GROWS):
                    pltpu.async_copy(
                        x_hbm.at[i1.at[kk]], buf.at[pl.ds(kk * _GSUB, _GSUB)], sem
                    )
                for kk in range(_GROWS):
                    pltpu.make_async_copy(
                        x_hbm.at[i1.at[kk]], buf.at[pl.ds(kk * _GSUB, _GSUB)], sem
                    ).wait()
                for kk in range(_GROWS):
                    pltpu.async_copy(
                        xn_hbm.at[i0.at[kk]],
                        buf.at[pl.ds(kk * _GSUB, _GSUB)],
                        sem,
                        add=True,
                    )
                for kk in range(_GROWS):
                    pltpu.make_async_copy(
                        xn_hbm.at[i0.at[kk]], buf.at[pl.ds(kk * _GSUB, _GSUB)], sem
                    ).wait()
                pltpu.sync_copy(buf, out_hbm.at[pl.ds(c * _GCHUNK, _GCHUNK)])

            return 0

        lax.fori_loop(0, iters, chunk, 0)

    return k(x, xneg, e0r, e1r)


_ERB = 2560


def _tc_edge(diffs, e_feat, w1, w2, b):
    def body(d_ref, f_ref, w1_ref, w2_ref, b_ref, o_ref):
        d = d_ref[...]
        h = jnp.dot(d, w1_ref[...], preferred_element_type=jnp.float32)
        h = h + jnp.dot(f_ref[...], w2_ref[...], preferred_element_type=jnp.float32)
        h = h + b_ref[...]
        o_ref[...] = d + jnp.maximum(h, 0.0)

    return pl.pallas_call(
        body,
        grid=(_E // _ERB,),
        in_specs=[
            pl.BlockSpec((_ERB, _WIDTH), lambda i: (i, 0)),
            pl.BlockSpec((_ERB, _NFEAT), lambda i: (i, 0)),
            pl.BlockSpec((_WIDTH, _WIDTH), lambda i: (0, 0)),
            pl.BlockSpec((_NFEAT, _WIDTH), lambda i: (0, 0)),
            pl.BlockSpec((1, _WIDTH), lambda i: (0, 0)),
        ],
        out_specs=pl.BlockSpec((_ERB, _WIDTH), lambda i: (i, 0)),
        out_shape=jax.ShapeDtypeStruct((_E, _WIDTH), jnp.float32),
    )(diffs, e_feat, w1, w2, b)


_OWN = 320
_NPAD = _OWN * _NW
_SCHUNK = 3200
_SVEC = _SCHUNK // 16
_SROWS = _SCHUNK // 16 // 5
_BATCH = 256
_NEG = float("-inf")


def _sc_segmax(e1, ef):
    mesh = plsc.VectorSubcoreMesh(core_axis_name="c", subcore_axis_name="s")

    @functools.partial(
        pl.kernel,
        mesh=mesh,
        compiler_params=pltpu.CompilerParams(needs_layout_passes=False),
        out_type=jax.ShapeDtypeStruct((_NPAD, _WIDTH), jnp.float32),
        scratch_types=[
            pltpu.VMEM((_SCHUNK,), jnp.int32),
            pltpu.VMEM((_SCHUNK + 96,), jnp.int32),
            pltpu.VMEM((2, 128), jnp.int32),
            pltpu.VMEM((_BATCH,), jnp.int32),
            pltpu.VMEM((_BATCH, _WIDTH), jnp.float32),
            pltpu.VMEM((_OWN, _WIDTH), jnp.float32),
            pltpu.VMEM((16,), jnp.int32),
            pltpu.VMEM_SHARED((_SCHUNK, _WIDTH), jnp.float32),
            pltpu.SemaphoreType.DMA,
        ],
    )
    def k(e1_hbm, ef_hbm, out_hbm, idxc, mylist, idsb, lib, rowbuf, acc,
          tmp16, spbuf, sem):
        sid = lax.axis_index("s")
        wid = sid * 2 + lax.axis_index("c")
        lo = wid * _OWN
        iota16 = lax.iota(jnp.int32, 16)
        neg = jnp.full((16,), _NEG, jnp.float32)

        def init_acc(i, _):
            for c in range(8):
                acc[i, pl.ds(c * 16, 16)] = neg
            return 0

        lax.fori_loop(0, _OWN, init_acc, 0)

        zero16 = jnp.zeros((16,), jnp.int32)

        def init_list(j, _):
            mylist[pl.ds(j * 16, 16)] = zero16
            return 0

        lax.fori_loop(0, (_SCHUNK + 96) // 16, init_list, 0)

        def chunk(t, _):
            plsc.subcore_barrier()
            for h in range(5):
                pltpu.async_copy(
                    ef_hbm.at[
                        pl.ds(t * _SCHUNK + (sid * 5 + h) * _SROWS, _SROWS)
                    ],
                    spbuf.at[pl.ds((sid * 5 + h) * _SROWS, _SROWS)],
                    sem,
                )
            pltpu.sync_copy(e1_hbm.at[pl.ds(t * _SCHUNK, _SCHUNK)], idxc)

            def scanb(j4, cv):
                vs, ms, incs = [], [], []
                for u in range(4):
                    v = idxc[pl.ds((j4 * 4 + u) * 16, 16)]
                    m = (v >= lo) & (v < lo + _OWN)
                    vs.append(v)
                    ms.append(m)
                    incs.append(jnp.cumsum(jnp.where(m, zero16 + 1, zero16)))
                for u in range(4):
                    pos = cv + incs[u] - 1
                    eid = (j4 * 4 + u) * 16 + iota16
                    packed = jnp.bitwise_or(
                        lax.shift_left(eid, 9), vs[u] - lo
                    )
                    plsc.store_scatter(mylist, [pos], packed, mask=ms[u])
                    cv = cv + plsc.all_reduce_population_count(ms[u])
                return cv

            cnt_vec = lax.fori_loop(0, _SVEC // 4, scanb, zero16)
            cnt = jnp.max(cnt_vec)
            for h in range(5):
                pltpu.make_async_copy(
                    ef_hbm.at[
                        pl.ds(t * _SCHUNK + (sid * 5 + h) * _SROWS, _SROWS)
                    ],
                    spbuf.at[pl.ds((sid * 5 + h) * _SROWS, _SROWS)],
                    sem,
                ).wait()
            plsc.subcore_barrier()

            def sub(s, _):
                for g in range(16):
                    pk = mylist[pl.ds(s * _BATCH + g * 16, 16)]
                    idsb[g // 8, pl.ds((g % 8) * 16, 16)] = (
                        lax.shift_right_logical(pk, 9)
                    )
                    lib[pl.ds(g * 16, 16)] = jnp.bitwise_and(pk, 511)
                pltpu.async_copy(
                    spbuf.at[idsb.at[0]], rowbuf.at[pl.ds(0, 128)], sem
                )
                pltpu.async_copy(
                    spbuf.at[idsb.at[1]], rowbuf.at[pl.ds(128, 128)], sem
                )
                pltpu.make_async_copy(
                    spbuf.at[idsb.at[0]], rowbuf.at[pl.ds(0, 128)], sem
                ).wait()
                pltpu.make_async_copy(
                    spbuf.at[idsb.at[1]], rowbuf.at[pl.ds(128, 128)], sem
                ).wait()
                nrows = jnp.clip(cnt - s * _BATCH, 0, _BATCH)
                ngroups = lax.shift_right_logical(nrows + 15, 4)

                def grp(g, _):
                    base = g * 16
                    li = lib[pl.ds(base, 16)]
                    li_s, perm = plsc.sort_key_val(li, iota16)
                    tmp16[pl.ds(0, 16)] = li_s
                    prev = plsc.load_gather(
                        tmp16, [jnp.maximum(iota16 - 1, 0)]
                    )
                    is_start = (li_s != prev) | (iota16 == 0)
                    run_start = plsc.cummax(
                        jnp.where(is_start, iota16, zero16)
                    )
                    run_pos = iota16 - run_start
                    valid = perm < (nrows - base)
                    nr = jnp.max(jnp.where(valid, run_pos, zero16)) + 1
                    rowi = base + perm

                    def rnd(r, _):
                        mr = (run_pos == r) & valid
                        for j in range(_WIDTH):
                            colj = zero16 + j
                            av = plsc.load_gather(acc, [li_s, colj], mask=mr)
                            rv = plsc.load_gather(rowbuf, [rowi, colj], mask=mr)
                            plsc.store_scatter(
                                acc, [li_s, colj], jnp.maximum(av, rv), mask=mr
                            )
                        return 0

                    lax.fori_loop(0, nr, rnd, 0)
                    return 0

                lax.fori_loop(0, ngroups, grp, 0)

            def sub_guard(s, _):
                pl.when(s * _BATCH < cnt)(lambda: sub(s, 0))
                return 0

            lax.fori_loop(0, _SCHUNK // _BATCH + 1, sub_guard, 0)
            return 0

        lax.fori_loop(0, _E // _SCHUNK, chunk, 0)

        zf = jnp.zeros((16,), jnp.float32)

        def fini(i, _):
            for c in range(8):
                v = acc[i, pl.ds(c * 16, 16)]
                acc[i, pl.ds(c * 16, 16)] = jnp.where(v == _NEG, zf, v)
            return 0

        lax.fori_loop(0, _OWN, fini, 0)
        pltpu.sync_copy(acc, out_hbm.at[pl.ds(lo, _OWN)])

    return k(e1, ef)


_NRB = 2000


def _tc_node(x, maxes, m1, m2, b):
    def body(x_ref, mx_ref, m1_ref, m2_ref, b_ref, o_ref):
        xv = x_ref[...]
        h = jnp.dot(xv, m1_ref[...], preferred_element_type=jnp.float32)
        h = h + jnp.dot(mx_ref[...], m2_ref[...], preferred_element_type=jnp.float32)
        h = h + b_ref[...]
        o_ref[...] = xv + jnp.maximum(h, 0.0)

    return pl.pallas_call(
        body,
        grid=(_N // _NRB,),
        in_specs=[
            pl.BlockSpec((_NRB, _WIDTH), lambda i: (i, 0)),
            pl.BlockSpec((_NRB, _WIDTH), lambda i: (i, 0)),
            pl.BlockSpec((_WIDTH, _WIDTH), lambda i: (0, 0)),
            pl.BlockSpec((_WIDTH, _WIDTH), lambda i: (0, 0)),
            pl.BlockSpec((1, _WIDTH), lambda i: (0, 0)),
        ],
        out_specs=pl.BlockSpec((_NRB, _WIDTH), lambda i: (i, 0)),
        out_shape=jax.ShapeDtypeStruct((_N, _WIDTH), jnp.float32),
    )(x, maxes, m1, m2, b)


def kernel(x, e, e_feat, edge_w, edge_b, mlp_w, mlp_b):
    e32 = e.astype(jnp.int32)
    e0r = e32[0].reshape(_E // _GSUB, _GSUB)
    e1r = e32[1].reshape(_E // _GSUB, _GSUB)
    diffs = _sc_diffs(x, -x, e0r, e1r)
    ef = _tc_edge(
        diffs, e_feat, edge_w[:_WIDTH], edge_w[_WIDTH:], edge_b.reshape(1, _WIDTH)
    )
    maxes = _sc_segmax(e32[1], ef)[:_N]
    return _tc_node(
        x, maxes, mlp_w[:_WIDTH], mlp_w[_WIDTH:], mlp_b.reshape(1, _WIDTH)
    )
